# Initial kernel scaffold; baseline (speedup 1.0000x reference)
import jax
import jax.numpy as jnp
from jax.experimental import pallas as pl


def kernel(x, xyz, Wq, gq, bq, Wk, gk, bk, Wv, Wpe1, gpe, bpe, Wpe2, Wwe1, gwe, bwe, Wwe2, Wo, go, bo):
    raise NotImplementedError("write your pallas kernel here")


# speedup vs baseline: 8.1463x; 8.1463x over previous
import functools

import jax
import jax.numpy as jnp
from jax import lax
from jax.experimental import pallas as pl
from jax.experimental.pallas import tpu as pltpu
from jax.experimental.pallas import tpu_sc as plsc

_HP = lax.Precision.HIGHEST

B, C, N, G, K = 2, 256, 4096, 8, 16
BN_ = B * N
S_ = BN_ * K
W_ = C + 128
NW = 32
SPW = S_ // NW
CH = 128

TN = 512
TQ = 512
TSA = 512
TP = 1024
TP2 = 256


def _first(*ids):
    f = ids[0] == 0
    for i in ids[1:]:
        f = jnp.logical_and(f, i == 0)
    return f


def _k1_body(x_ref, wq_ref, wk_ref, wv_ref, yq_ref, yk_ref, vt_ref, st_ref):
    b = pl.program_id(0)
    j = pl.program_id(1)
    x_t = x_ref[0]
    yq = jnp.dot(wq_ref[...], x_t, preferred_element_type=jnp.float32)
    yk = jnp.dot(wk_ref[...], x_t, preferred_element_type=jnp.float32)
    vt = lax.dot_general(x_t, wv_ref[...], (((0,), (1,)), ((), ())),
                         preferred_element_type=jnp.float32)
    yq_ref[0] = yq
    yk_ref[0] = yk
    vt_ref[...] = vt
    col = lax.broadcasted_iota(jnp.int32, (TN, 8), 1)
    p0 = (col == 0).astype(jnp.float32)
    p1 = (col == 1).astype(jnp.float32)
    p2 = (col == 2).astype(jnp.float32)
    p3 = (col == 3).astype(jnp.float32)
    part = (jnp.dot(yq, p0, precision=_HP, preferred_element_type=jnp.float32)
            + jnp.dot(yq * yq, p1, precision=_HP,
                      preferred_element_type=jnp.float32)
            + jnp.dot(yk, p2, precision=_HP,
                      preferred_element_type=jnp.float32)
            + jnp.dot(yk * yk, p3, precision=_HP,
                      preferred_element_type=jnp.float32))

    @pl.when(_first(b, j))
    def _():
        st_ref[...] = jnp.zeros_like(st_ref)

    st_ref[...] += part


def _k1(x, Wq, Wk, Wv):
    return pl.pallas_call(
        _k1_body,
        grid=(B, N // TN),
        in_specs=[
            pl.BlockSpec((1, C, TN), lambda b, j: (b, 0, j)),
            pl.BlockSpec((C, C), lambda b, j: (0, 0)),
            pl.BlockSpec((C, C), lambda b, j: (0, 0)),
            pl.BlockSpec((C, C), lambda b, j: (0, 0)),
        ],
        out_specs=[
            pl.BlockSpec((1, C, TN), lambda b, j: (b, 0, j)),
            pl.BlockSpec((1, C, TN), lambda b, j: (b, 0, j)),
            pl.BlockSpec((TN, C), lambda b, j: (b * (N // TN) + j, 0)),
            pl.BlockSpec((C, 8), lambda b, j: (0, 0)),
        ],
        out_shape=[
            jax.ShapeDtypeStruct((B, C, N), jnp.float32),
            jax.ShapeDtypeStruct((B, C, N), jnp.float32),
            jax.ShapeDtypeStruct((BN_, C), jnp.float32),
            jax.ShapeDtypeStruct((C, 8), jnp.float32),
        ],
    )(x, Wq, Wk, Wv)


def _k2_body(ptsq_ref, ptsa_ref, idx_ref):
    b = pl.program_id(0)
    qt = ptsq_ref[0]
    ap = ptsa_ref[0]
    sq_q = jnp.sum(qt * qt, axis=1, keepdims=True)
    sq_ac = jnp.sum(ap * ap, axis=1, keepdims=True)
    sq_a = lax.dot_general(jnp.ones((1, 1), jnp.float32), sq_ac,
                           (((1,), (1,)), ((), ())),
                           precision=lax.Precision.HIGHEST,
                           preferred_element_type=jnp.float32)
    cross = lax.dot_general(qt.astype(jnp.bfloat16),
                            ap.astype(jnp.bfloat16),
                            (((1,), (1,)), ((), ())),
                            preferred_element_type=jnp.float32)
    d = sq_q + sq_a - 2.0 * cross
    iota_n = lax.broadcasted_iota(jnp.int32, (TQ, N), 1)
    lane16 = lax.broadcasted_iota(jnp.int32, (TQ, K), 1)
    acc = jnp.zeros((TQ, K), jnp.int32)
    big = jnp.float32(jnp.inf)
    for j in range(K):
        m = jnp.min(d, axis=1, keepdims=True)
        am = jnp.min(jnp.where(d == m, iota_n, N), axis=1, keepdims=True)
        acc = jnp.where(lane16 == j, am + b * N, acc)
        d = jnp.where(iota_n == am, big, d)
    idx_ref[...] = acc


def _k2(ptsT):
    return pl.pallas_call(
        _k2_body,
        grid=(B, N // TQ),
        in_specs=[
            pl.BlockSpec((1, TQ, 3), lambda b, j: (b, j, 0)),
            pl.BlockSpec((1, N, 3), lambda b, j: (b, 0, 0)),
        ],
        out_specs=pl.BlockSpec((TQ, K), lambda b, j: (b * (N // TQ) + j, 0)),
        out_shape=jax.ShapeDtypeStruct((BN_, K), jnp.int32),
    )(ptsT, ptsT)


def _k3_body(yq_ref, yk_ref, xyzt_ref, vt_ref, wwe1_ref, aq_ref, cq_ref,
             ak_ref, ck_ref, t1_ref, t2_ref):
    q = jnp.maximum(yq_ref[0] * aq_ref[...] + cq_ref[...], 0.0)
    k = jnp.maximum(yk_ref[0] * ak_ref[...] + ck_ref[...], 0.0)
    zq = lax.dot_general(q, wwe1_ref[...], (((0,), (1,)), ((), ())),
                         preferred_element_type=jnp.float32)
    zk = lax.dot_general(k, wwe1_ref[...], (((0,), (1,)), ((), ())),
                         preferred_element_type=jnp.float32)
    xyz_t = xyzt_ref[0]
    r3 = lax.broadcasted_iota(jnp.int32, (3, 128), 0)
    c3 = lax.broadcasted_iota(jnp.int32, (3, 128), 1)
    p3 = (r3 == c3).astype(jnp.float32)
    r8 = lax.broadcasted_iota(jnp.int32, (G, 128), 0)
    c8 = lax.broadcasted_iota(jnp.int32, (G, 128), 1)
    p8 = (c8 == r8 + 3).astype(jnp.float32)
    xp = jnp.dot(xyz_t, p3, precision=_HP, preferred_element_type=jnp.float32)
    t1_ref[:, 0:C] = vt_ref[...]
    t1_ref[:, C:W_] = xp + jnp.dot(zk, p8, precision=_HP,
                                   preferred_element_type=jnp.float32)
    t2_ref[...] = (xp + jnp.dot(zq, p8, precision=_HP,
                                preferred_element_type=jnp.float32))[:, 0:16]


def _k3(yq, yk, xyzT, vT, Wwe1, aq, cq, ak, ck):
    return pl.pallas_call(
        _k3_body,
        grid=(B, N // TN),
        in_specs=[
            pl.BlockSpec((1, C, TN), lambda b, j: (b, 0, j)),
            pl.BlockSpec((1, C, TN), lambda b, j: (b, 0, j)),
            pl.BlockSpec((1, TN, 3), lambda b, j: (b, j, 0)),
            pl.BlockSpec((TN, C), lambda b, j: (b * (N // TN) + j, 0)),
            pl.BlockSpec((G, C), lambda b, j: (0, 0)),
            pl.BlockSpec((C, 1), lambda b, j: (0, 0)),
            pl.BlockSpec((C, 1), lambda b, j: (0, 0)),
            pl.BlockSpec((C, 1), lambda b, j: (0, 0)),
            pl.BlockSpec((C, 1), lambda b, j: (0, 0)),
        ],
        out_specs=[
            pl.BlockSpec((TN, W_), lambda b, j: (b * (N // TN) + j, 0)),
            pl.BlockSpec((TN, 16), lambda b, j: (b * (N // TN) + j, 0)),
        ],
        out_shape=[
            jax.ShapeDtypeStruct((BN_, W_), jnp.float32),
            jax.ShapeDtypeStruct((BN_, 16), jnp.float32),
        ],
    )(yq, yk, xyzT, vT, Wwe1, aq, cq, ak, ck)


def _scg(T1, idxf):
    mesh = plsc.VectorSubcoreMesh(core_axis_name="c", subcore_axis_name="s")

    @functools.partial(
        pl.kernel,
        out_type=jax.ShapeDtypeStruct((S_, W_), jnp.float32),
        mesh=mesh,
        scratch_types=[
            pltpu.VMEM((CH,), jnp.int32),
            pltpu.VMEM((CH, W_), jnp.float32),
            pltpu.SemaphoreType.DMA,
        ],
    )
    def scg(t1_hbm, idx_hbm, out_hbm, idx_c, rows_v, sem):
        wid = lax.axis_index("s") * 2 + lax.axis_index("c")
        base_s = wid * SPW

        def body_c(ci, _):
            off = base_s + ci * CH
            pltpu.sync_copy(idx_hbm.at[pl.ds(off, CH)], idx_c)
            pltpu.async_copy(t1_hbm.at[idx_c], rows_v, sem).wait()
            pltpu.sync_copy(rows_v, out_hbm.at[pl.ds(off, CH)])
            return 0

        lax.fori_loop(0, SPW // CH, body_c, 0)

    return scg(T1, idxf)


def _k4a_body(g_ref, t2_ref, st_ref):
    i = pl.program_id(0)
    cen = t2_ref[:, 0, 0:3]
    su = jnp.zeros((1, 3), jnp.float32)
    outer = jnp.zeros((3, 3), jnp.float32)
    for kk in range(K):
        rel = g_ref[:, kk, 0:3] - cen
        su = su + lax.dot_general(jnp.ones((8, TSA), jnp.float32), rel,
                                  (((1,), (0,)), ((), ())),
                                  precision=_HP,
                                  preferred_element_type=jnp.float32)[0:1]
        outer = outer + lax.dot_general(rel, rel, (((0,), (0,)), ((), ())),
                                        precision=_HP,
                                        preferred_element_type=jnp.float32)
    r8 = lax.broadcasted_iota(jnp.int32, (8, 1), 0)
    a0 = (r8 == 0).astype(jnp.float32)
    rb = lax.broadcasted_iota(jnp.int32, (8, 3), 0)
    cb = lax.broadcasted_iota(jnp.int32, (8, 3), 1)
    bsel = (rb == cb + 1).astype(jnp.float32)
    r38 = lax.broadcasted_iota(jnp.int32, (3, 8), 0)
    c38 = lax.broadcasted_iota(jnp.int32, (3, 8), 1)
    p38 = (r38 == c38).astype(jnp.float32)
    part = jnp.dot(jnp.dot(a0, su, precision=_HP,
                           preferred_element_type=jnp.float32)
                   + jnp.dot(bsel, outer, precision=_HP,
                             preferred_element_type=jnp.float32),
                   p38, precision=_HP,
                   preferred_element_type=jnp.float32)

    @pl.when(i == 0)
    def _():
        st_ref[...] = jnp.zeros_like(st_ref)

    st_ref[...] += part


def _k4a(g3, t23):
    return pl.pallas_call(
        _k4a_body,
        grid=(BN_ // TSA,),
        in_specs=[
            pl.BlockSpec((TSA, K, 128), lambda i: (i, 0, 2)),
            pl.BlockSpec((TSA, 1, 16), lambda i: (i, 0, 0)),
        ],
        out_specs=pl.BlockSpec((8, 8), lambda i: (0, 0)),
        out_shape=jax.ShapeDtypeStruct((8, 8), jnp.float32),
    )(g3, t23)


def _k4_body(g_ref, t2_ref, wpe1_ref, ape_ref, cpe_ref, m_ref, z_ref, st_ref):
    i = pl.program_id(0)
    cen3 = t2_ref[:, 0, 0:3]
    cen8 = t2_ref[:, 0, 3:11]
    su = jnp.zeros((1, 8), jnp.float32)
    sq = jnp.zeros((1, 8), jnp.float32)
    for kk in range(K):
        rel = g_ref[:, kk, 0:3] - cen3
        zdiff = g_ref[:, kk, 3:11] - cen8
        pe1 = lax.dot_general(rel, wpe1_ref[...], (((1,), (1,)), ((), ())),
                              preferred_element_type=jnp.float32)
        h = jnp.maximum(pe1 * ape_ref[...] + cpe_ref[...], 0.0)
        zpe = lax.dot_general(h, m_ref[...], (((1,), (1,)), ((), ())),
                              preferred_element_type=jnp.float32)
        z = zdiff + zpe
        z_ref[:, kk, :] = z
        su = su + jnp.sum(z, axis=0, keepdims=True)
        sq = sq + jnp.sum(z * z, axis=0, keepdims=True)
    r8 = lax.broadcasted_iota(jnp.int32, (8, 1), 0)
    a0 = (r8 == 0).astype(jnp.float32)
    a1 = (r8 == 1).astype(jnp.float32)
    part = (jnp.dot(a0, su, precision=_HP,
                    preferred_element_type=jnp.float32)
            + jnp.dot(a1, sq, precision=_HP,
                      preferred_element_type=jnp.float32))

    @pl.when(i == 0)
    def _():
        st_ref[...] = jnp.zeros_like(st_ref)

    st_ref[...] += part


def _k4(g3, t23, Wpe1, ape, cpe, M):
    return pl.pallas_call(
        _k4_body,
        grid=(BN_ // TSA,),
        in_specs=[
            pl.BlockSpec((TSA, K, 128), lambda i: (i, 0, 2)),
            pl.BlockSpec((TSA, 1, 16), lambda i: (i, 0, 0)),
            pl.BlockSpec((C, 3), lambda i: (0, 0)),
            pl.BlockSpec((1, C), lambda i: (0, 0)),
            pl.BlockSpec((1, C), lambda i: (0, 0)),
            pl.BlockSpec((G, C), lambda i: (0, 0)),
        ],
        out_specs=[
            pl.BlockSpec((TSA, K, G), lambda i: (i, 0, 0)),
            pl.BlockSpec((8, 8), lambda i: (0, 0)),
        ],
        out_shape=[
            jax.ShapeDtypeStruct((BN_, K, G), jnp.float32),
            jax.ShapeDtypeStruct((8, 8), jnp.float32),
        ],
    )(g3, t23, Wpe1, ape, cpe, M)


def _k5_body(z_ref, az_ref, cz_ref, wblk_ref, t8_ref, w_ref):
    a = jnp.maximum(z_ref[...] * az_ref[...] + cz_ref[...], 0.0)
    l = jnp.dot(a, wblk_ref[...], preferred_element_type=jnp.float32)
    m8 = l[:, 0:G]
    for kk in range(1, K):
        m8 = jnp.maximum(m8, l[:, kk * G:(kk + 1) * G])
    e = jnp.exp(l - jnp.dot(m8, t8_ref[...], precision=_HP,
                            preferred_element_type=jnp.float32))
    s8 = e[:, 0:G]
    for kk in range(1, K):
        s8 = s8 + e[:, kk * G:(kk + 1) * G]
    w_ref[...] = e / jnp.dot(s8, t8_ref[...], precision=_HP,
                             preferred_element_type=jnp.float32)


def _k5(z2, az, cz, Wblk, T8):
    return pl.pallas_call(
        _k5_body,
        grid=(BN_ // TP,),
        in_specs=[
            pl.BlockSpec((TP, K * G), lambda i: (i, 0)),
            pl.BlockSpec((1, K * G), lambda i: (0, 0)),
            pl.BlockSpec((1, K * G), lambda i: (0, 0)),
            pl.BlockSpec((K * G, K * G), lambda i: (0, 0)),
            pl.BlockSpec((G, K * G), lambda i: (0, 0)),
        ],
        out_specs=pl.BlockSpec((TP, K * G), lambda i: (i, 0)),
        out_shape=jax.ShapeDtypeStruct((BN_, K * G), jnp.float32),
    )(z2, az, cz, Wblk, T8)


def _k6_body(g_ref, t2_ref, w_ref, wpe1_ref, ape_ref, cpe_ref, wpe2_ref,
             e_ref, wo_ref, yo_ref, st_ref):
    i = pl.program_id(0)
    cen3 = t2_ref[:, 0, 0:3]
    acc = jnp.zeros((TP2, C), jnp.float32)
    for kk in range(K):
        vk = g_ref[:, kk, 0:C]
        relk = g_ref[:, kk, C:C + 3] - cen3
        wk = w_ref[:, kk * G:(kk + 1) * G]
        wek = jnp.dot(wk, e_ref[...], precision=_HP,
                      preferred_element_type=jnp.float32)
        pe1 = lax.dot_general(relk, wpe1_ref[...], (((1,), (1,)), ((), ())),
                              preferred_element_type=jnp.float32)
        h = jnp.maximum(pe1 * ape_ref[...] + cpe_ref[...], 0.0)
        pek = lax.dot_general(h, wpe2_ref[...], (((1,), (1,)), ((), ())),
                              preferred_element_type=jnp.float32)
        acc = acc + (vk + pek) * wek
    yo = lax.dot_general(acc, wo_ref[...], (((1,), (1,)), ((), ())),
                         preferred_element_type=jnp.float32)
    yo_ref[...] = yo
    su = jnp.sum(yo, axis=0, keepdims=True)
    sq = jnp.sum(yo * yo, axis=0, keepdims=True)
    r8 = lax.broadcasted_iota(jnp.int32, (8, 1), 0)
    a0 = (r8 == 0).astype(jnp.float32)
    a1 = (r8 == 1).astype(jnp.float32)
    part = (jnp.dot(a0, su, precision=_HP,
                    preferred_element_type=jnp.float32)
            + jnp.dot(a1, sq, precision=_HP,
                      preferred_element_type=jnp.float32))

    @pl.when(i == 0)
    def _():
        st_ref[...] = jnp.zeros_like(st_ref)

    st_ref[...] += part


def _k6(g3, t23, w2, Wpe1, ape, cpe, Wpe2, E, Wo):
    return pl.pallas_call(
        _k6_body,
        grid=(BN_ // TP2,),
        in_specs=[
            pl.BlockSpec((TP2, K, W_), lambda i: (i, 0, 0)),
            pl.BlockSpec((TP2, 1, 16), lambda i: (i, 0, 0)),
            pl.BlockSpec((TP2, K * G), lambda i: (i, 0)),
            pl.BlockSpec((C, 3), lambda i: (0, 0)),
            pl.BlockSpec((1, C), lambda i: (0, 0)),
            pl.BlockSpec((1, C), lambda i: (0, 0)),
            pl.BlockSpec((C, C), lambda i: (0, 0)),
            pl.BlockSpec((G, C), lambda i: (0, 0)),
            pl.BlockSpec((C, C), lambda i: (0, 0)),
        ],
        out_specs=[
            pl.BlockSpec((TP2, C), lambda i: (i, 0)),
            pl.BlockSpec((8, C), lambda i: (0, 0)),
        ],
        out_shape=[
            jax.ShapeDtypeStruct((BN_, C), jnp.float32),
            jax.ShapeDtypeStruct((8, C), jnp.float32),
        ],
    )(g3, t23, w2, Wpe1, ape, cpe, Wpe2, E, Wo)


def _k7_body(yo_ref, ao_ref, co_ref, out_ref):
    yo = yo_ref[...]
    r = lax.broadcasted_iota(jnp.int32, (C, C), 0)
    c = lax.broadcasted_iota(jnp.int32, (C, C), 1)
    ident = (r == c).astype(jnp.float32)
    t = lax.dot_general(ident, yo, (((1,), (1,)), ((), ())),
                        precision=_HP,
                        preferred_element_type=jnp.float32)
    out_ref[0] = t * ao_ref[...] + co_ref[...]


def _k7(yoT, ao, co):
    return pl.pallas_call(
        _k7_body,
        grid=(B, N // TN),
        in_specs=[
            pl.BlockSpec((TN, C), lambda b, j: (b * (N // TN) + j, 0)),
            pl.BlockSpec((C, 1), lambda b, j: (0, 0)),
            pl.BlockSpec((C, 1), lambda b, j: (0, 0)),
        ],
        out_specs=pl.BlockSpec((1, C, TN), lambda b, j: (b, 0, j)),
        out_shape=jax.ShapeDtypeStruct((B, C, N), jnp.float32),
    )(yoT, ao, co)


def _bn_affine(s_sum, s_sq, n, gamma, beta):
    mu = s_sum / n
    var = s_sq / n - mu * mu
    inv = 1.0 / jnp.sqrt(var + 1e-5)
    a = gamma * inv
    c = beta - mu * a
    return a, c


@jax.jit
def kernel(x, xyz, Wq, gq, bq, Wk, gk, bk, Wv, Wpe1, gpe, bpe, Wpe2,
           Wwe1, gwe, bwe, Wwe2, Wo, go, bo):
    ptsT = jnp.transpose(xyz, (0, 2, 1))

    yq, yk, vT, st1 = _k1(x, Wq, Wk, Wv)
    idx = _k2(ptsT)
    idxf = idx.reshape(S_)

    aq, cq = _bn_affine(st1[:, 0], st1[:, 1], BN_, gq, bq)
    ak, ck = _bn_affine(st1[:, 2], st1[:, 3], BN_, gk, bk)

    T1, T2 = _k3(yq, yk, ptsT, vT, Wwe1,
                 aq.reshape(C, 1), cq.reshape(C, 1),
                 ak.reshape(C, 1), ck.reshape(C, 1))

    Gt = _scg(T1, idxf)
    g3 = Gt.reshape(BN_, K, W_)
    t23 = T2.reshape(BN_, 1, 16)

    st_rel = _k4a(g3, t23)
    mu_rel = st_rel[0, 0:3] / S_
    S_rel = st_rel[1:4, 0:3] / S_
    cov = S_rel - jnp.outer(mu_rel, mu_rel)
    mu_pe1 = Wpe1 @ mu_rel
    var_pe1 = jnp.sum((Wpe1 @ cov) * Wpe1, axis=1)
    inv_pe = 1.0 / jnp.sqrt(var_pe1 + 1e-5)
    ape = (gpe * inv_pe).reshape(1, C)
    cpe = (bpe - mu_pe1 * gpe * inv_pe).reshape(1, C)

    M = Wwe1 @ Wpe2
    z, st_z = _k4(g3, t23, Wpe1, ape, cpe, M)

    az, cz = _bn_affine(st_z[0, :], st_z[1, :], S_, gwe, bwe)
    az128 = jnp.tile(az, K).reshape(1, K * G)
    cz128 = jnp.tile(cz, K).reshape(1, K * G)
    Wblk = jnp.kron(jnp.eye(K, dtype=jnp.float32), Wwe2.T)
    T8 = jnp.kron(jnp.ones((1, K), jnp.float32), jnp.eye(G, dtype=jnp.float32))

    z2 = z.reshape(BN_, K * G)
    w2 = _k5(z2, az128, cz128, Wblk, T8)

    E = jnp.kron(jnp.eye(G, dtype=jnp.float32),
                 jnp.ones((1, C // G), jnp.float32))
    yoT, st_o = _k6(g3, t23, w2, Wpe1, ape, cpe, Wpe2, E, Wo)

    ao, co = _bn_affine(st_o[0, :], st_o[1, :], BN_, go, bo)
    return _k7(yoT, ao.reshape(C, 1), co.reshape(C, 1))


# speedup vs baseline: 9.9931x; 1.2267x over previous
import functools

import jax
import jax.numpy as jnp
from jax import lax
from jax.experimental import pallas as pl
from jax.experimental.pallas import tpu as pltpu
from jax.experimental.pallas import tpu_sc as plsc

_HP = lax.Precision.HIGHEST

B, C, N, G, K = 2, 256, 4096, 8, 16
BN_ = B * N
S_ = BN_ * K
W_ = C + 128
NW = 32
SPW = S_ // NW
CH = 128

TN = 512
TQ = 512
TSA = 512
TP = 1024
TP2 = 256


def _first(*ids):
    f = ids[0] == 0
    for i in ids[1:]:
        f = jnp.logical_and(f, i == 0)
    return f


def _k1_body(x_ref, wq_ref, wk_ref, wv_ref, yq_ref, yk_ref, vt_ref, st_ref):
    b = pl.program_id(0)
    j = pl.program_id(1)
    x_t = x_ref[0]
    yq = jnp.dot(wq_ref[...], x_t, preferred_element_type=jnp.float32)
    yk = jnp.dot(wk_ref[...], x_t, preferred_element_type=jnp.float32)
    vt = lax.dot_general(x_t, wv_ref[...], (((0,), (1,)), ((), ())),
                         preferred_element_type=jnp.float32)
    yq_ref[0] = yq
    yk_ref[0] = yk
    vt_ref[...] = vt
    col = lax.broadcasted_iota(jnp.int32, (TN, 8), 1)
    p0 = (col == 0).astype(jnp.float32)
    p1 = (col == 1).astype(jnp.float32)
    p2 = (col == 2).astype(jnp.float32)
    p3 = (col == 3).astype(jnp.float32)
    part = (jnp.dot(yq, p0, precision=_HP, preferred_element_type=jnp.float32)
            + jnp.dot(yq * yq, p1, precision=_HP,
                      preferred_element_type=jnp.float32)
            + jnp.dot(yk, p2, precision=_HP,
                      preferred_element_type=jnp.float32)
            + jnp.dot(yk * yk, p3, precision=_HP,
                      preferred_element_type=jnp.float32))

    @pl.when(_first(b, j))
    def _():
        st_ref[...] = jnp.zeros_like(st_ref)

    st_ref[...] += part


def _k1(x, Wq, Wk, Wv):
    return pl.pallas_call(
        _k1_body,
        grid=(B, N // TN),
        in_specs=[
            pl.BlockSpec((1, C, TN), lambda b, j: (b, 0, j)),
            pl.BlockSpec((C, C), lambda b, j: (0, 0)),
            pl.BlockSpec((C, C), lambda b, j: (0, 0)),
            pl.BlockSpec((C, C), lambda b, j: (0, 0)),
        ],
        out_specs=[
            pl.BlockSpec((1, C, TN), lambda b, j: (b, 0, j)),
            pl.BlockSpec((1, C, TN), lambda b, j: (b, 0, j)),
            pl.BlockSpec((TN, C), lambda b, j: (b * (N // TN) + j, 0)),
            pl.BlockSpec((C, 8), lambda b, j: (0, 0)),
        ],
        out_shape=[
            jax.ShapeDtypeStruct((B, C, N), jnp.float32),
            jax.ShapeDtypeStruct((B, C, N), jnp.float32),
            jax.ShapeDtypeStruct((BN_, C), jnp.float32),
            jax.ShapeDtypeStruct((C, 8), jnp.float32),
        ],
    )(x, Wq, Wk, Wv)


def _k2_body(ptsq_ref, ptsa_ref, idx_ref):
    b = pl.program_id(0)
    qt = ptsq_ref[0]
    ap = ptsa_ref[0]
    sq_q = jnp.sum(qt * qt, axis=1, keepdims=True)
    sq_ac = jnp.sum(ap * ap, axis=1, keepdims=True)
    sq_a = lax.dot_general(jnp.ones((1, 1), jnp.float32), sq_ac,
                           (((1,), (1,)), ((), ())),
                           precision=lax.Precision.HIGHEST,
                           preferred_element_type=jnp.float32)
    cross = lax.dot_general(qt.astype(jnp.bfloat16),
                            ap.astype(jnp.bfloat16),
                            (((1,), (1,)), ((), ())),
                            preferred_element_type=jnp.float32)
    d = sq_q + sq_a - 2.0 * cross
    iota_n = lax.broadcasted_iota(jnp.int32, (TQ, N), 1)
    lane16 = lax.broadcasted_iota(jnp.int32, (TQ, K), 1)
    acc = jnp.zeros((TQ, K), jnp.int32)
    big = jnp.float32(jnp.inf)
    for j in range(K):
        am = jnp.argmin(d, axis=1, keepdims=True).astype(jnp.int32)
        acc = jnp.where(lane16 == j, am + b * N, acc)
        d = jnp.where(iota_n == am, big, d)
    idx_ref[...] = acc


def _k2(ptsT):
    return pl.pallas_call(
        _k2_body,
        grid=(B, N // TQ),
        in_specs=[
            pl.BlockSpec((1, TQ, 3), lambda b, j: (b, j, 0)),
            pl.BlockSpec((1, N, 3), lambda b, j: (b, 0, 0)),
        ],
        out_specs=pl.BlockSpec((TQ, K), lambda b, j: (b * (N // TQ) + j, 0)),
        out_shape=jax.ShapeDtypeStruct((BN_, K), jnp.int32),
    )(ptsT, ptsT)


def _k3_body(yq_ref, yk_ref, xyzt_ref, wwe1_ref, aq_ref, cq_ref,
             ak_ref, ck_ref, t1_ref, t2_ref):
    q = jnp.maximum(yq_ref[0] * aq_ref[...] + cq_ref[...], 0.0)
    k = jnp.maximum(yk_ref[0] * ak_ref[...] + ck_ref[...], 0.0)
    zq = lax.dot_general(q, wwe1_ref[...], (((0,), (1,)), ((), ())),
                         preferred_element_type=jnp.float32)
    zk = lax.dot_general(k, wwe1_ref[...], (((0,), (1,)), ((), ())),
                         preferred_element_type=jnp.float32)
    xyz_t = xyzt_ref[0]
    r3 = lax.broadcasted_iota(jnp.int32, (3, 128), 0)
    c3 = lax.broadcasted_iota(jnp.int32, (3, 128), 1)
    p3 = (r3 == c3).astype(jnp.float32)
    r8 = lax.broadcasted_iota(jnp.int32, (G, 128), 0)
    c8 = lax.broadcasted_iota(jnp.int32, (G, 128), 1)
    p8 = (c8 == r8 + 3).astype(jnp.float32)
    xp = jnp.dot(xyz_t, p3, precision=_HP, preferred_element_type=jnp.float32)
    t1_ref[...] = xp + jnp.dot(zk, p8, precision=_HP,
                               preferred_element_type=jnp.float32)
    t2_ref[...] = (xp + jnp.dot(zq, p8, precision=_HP,
                                preferred_element_type=jnp.float32))[:, 0:16]


def _k3(yq, yk, xyzT, Wwe1, aq, cq, ak, ck):
    return pl.pallas_call(
        _k3_body,
        grid=(B, N // TN),
        in_specs=[
            pl.BlockSpec((1, C, TN), lambda b, j: (b, 0, j)),
            pl.BlockSpec((1, C, TN), lambda b, j: (b, 0, j)),
            pl.BlockSpec((1, TN, 3), lambda b, j: (b, j, 0)),
            pl.BlockSpec((G, C), lambda b, j: (0, 0)),
            pl.BlockSpec((C, 1), lambda b, j: (0, 0)),
            pl.BlockSpec((C, 1), lambda b, j: (0, 0)),
            pl.BlockSpec((C, 1), lambda b, j: (0, 0)),
            pl.BlockSpec((C, 1), lambda b, j: (0, 0)),
        ],
        out_specs=[
            pl.BlockSpec((TN, 128), lambda b, j: (b * (N // TN) + j, 0)),
            pl.BlockSpec((TN, 16), lambda b, j: (b * (N // TN) + j, 0)),
        ],
        out_shape=[
            jax.ShapeDtypeStruct((BN_, 128), jnp.float32),
            jax.ShapeDtypeStruct((BN_, 16), jnp.float32),
        ],
    )(yq, yk, xyzT, Wwe1, aq, cq, ak, ck)


def _sc_gather(table, idxf, width):
    mesh = plsc.VectorSubcoreMesh(core_axis_name="c", subcore_axis_name="s")

    @functools.partial(
        pl.kernel,
        out_type=jax.ShapeDtypeStruct((S_, width), jnp.float32),
        mesh=mesh,
        scratch_types=[
            pltpu.VMEM((2, CH), jnp.int32),
            pltpu.VMEM((2, CH, width), jnp.float32),
            pltpu.SemaphoreType.DMA,
            pltpu.SemaphoreType.DMA,
        ],
    )
    def scg(t_hbm, idx_hbm, out_hbm, idx2, rows2, sem0, sem1):
        wid = lax.axis_index("s") * 2 + lax.axis_index("c")
        base_s = wid * SPW
        sems = [sem0, sem1]

        def body_c(j, _):
            handles = []
            for bb in range(2):
                off = base_s + (2 * j + bb) * CH
                pltpu.sync_copy(idx_hbm.at[pl.ds(off, CH)], idx2.at[bb])
                handles.append(
                    pltpu.async_copy(t_hbm.at[idx2.at[bb]], rows2.at[bb],
                                     sems[bb]))
            for bb in range(2):
                off = base_s + (2 * j + bb) * CH
                handles[bb].wait()
                pltpu.sync_copy(rows2.at[bb], out_hbm.at[pl.ds(off, CH)])
            return 0

        lax.fori_loop(0, SPW // (2 * CH), body_c, 0)

    return scg(table, idxf)


def _k4a_body(g_ref, t2_ref, st_ref):
    i = pl.program_id(0)
    cen = t2_ref[:, 0, 0:3]
    su = jnp.zeros((1, 3), jnp.float32)
    outer = jnp.zeros((3, 3), jnp.float32)
    for kk in range(K):
        rel = g_ref[:, kk, 0:3] - cen
        su = su + lax.dot_general(jnp.ones((8, TSA), jnp.float32), rel,
                                  (((1,), (0,)), ((), ())),
                                  precision=_HP,
                                  preferred_element_type=jnp.float32)[0:1]
        outer = outer + lax.dot_general(rel, rel, (((0,), (0,)), ((), ())),
                                        precision=_HP,
                                        preferred_element_type=jnp.float32)
    r8 = lax.broadcasted_iota(jnp.int32, (8, 1), 0)
    a0 = (r8 == 0).astype(jnp.float32)
    rb = lax.broadcasted_iota(jnp.int32, (8, 3), 0)
    cb = lax.broadcasted_iota(jnp.int32, (8, 3), 1)
    bsel = (rb == cb + 1).astype(jnp.float32)
    r38 = lax.broadcasted_iota(jnp.int32, (3, 8), 0)
    c38 = lax.broadcasted_iota(jnp.int32, (3, 8), 1)
    p38 = (r38 == c38).astype(jnp.float32)
    part = jnp.dot(jnp.dot(a0, su, precision=_HP,
                           preferred_element_type=jnp.float32)
                   + jnp.dot(bsel, outer, precision=_HP,
                             preferred_element_type=jnp.float32),
                   p38, precision=_HP,
                   preferred_element_type=jnp.float32)

    @pl.when(i == 0)
    def _():
        st_ref[...] = jnp.zeros_like(st_ref)

    st_ref[...] += part


def _k4a(g3, t23):
    return pl.pallas_call(
        _k4a_body,
        grid=(BN_ // TSA,),
        in_specs=[
            pl.BlockSpec((TSA, K, 128), lambda i: (i, 0, 0)),
            pl.BlockSpec((TSA, 1, 16), lambda i: (i, 0, 0)),
        ],
        out_specs=pl.BlockSpec((8, 8), lambda i: (0, 0)),
        out_shape=jax.ShapeDtypeStruct((8, 8), jnp.float32),
    )(g3, t23)


def _k4_body(g_ref, t2_ref, wpe1_ref, ape_ref, cpe_ref, m_ref, z_ref, st_ref):
    i = pl.program_id(0)
    cen3 = t2_ref[:, 0, 0:3]
    cen8 = t2_ref[:, 0, 3:11]
    su = jnp.zeros((1, 8), jnp.float32)
    sq = jnp.zeros((1, 8), jnp.float32)
    for kk in range(K):
        rel = g_ref[:, kk, 0:3] - cen3
        zdiff = g_ref[:, kk, 3:11] - cen8
        pe1 = lax.dot_general(rel, wpe1_ref[...], (((1,), (1,)), ((), ())),
                              preferred_element_type=jnp.float32)
        h = jnp.maximum(pe1 * ape_ref[...] + cpe_ref[...], 0.0)
        zpe = lax.dot_general(h, m_ref[...], (((1,), (1,)), ((), ())),
                              preferred_element_type=jnp.float32)
        z = zdiff + zpe
        z_ref[:, kk, :] = z
        su = su + jnp.sum(z, axis=0, keepdims=True)
        sq = sq + jnp.sum(z * z, axis=0, keepdims=True)
    r8 = lax.broadcasted_iota(jnp.int32, (8, 1), 0)
    a0 = (r8 == 0).astype(jnp.float32)
    a1 = (r8 == 1).astype(jnp.float32)
    part = (jnp.dot(a0, su, precision=_HP,
                    preferred_element_type=jnp.float32)
            + jnp.dot(a1, sq, precision=_HP,
                      preferred_element_type=jnp.float32))

    @pl.when(i == 0)
    def _():
        st_ref[...] = jnp.zeros_like(st_ref)

    st_ref[...] += part


def _k4(g3, t23, Wpe1, ape, cpe, M):
    return pl.pallas_call(
        _k4_body,
        grid=(BN_ // TSA,),
        in_specs=[
            pl.BlockSpec((TSA, K, 128), lambda i: (i, 0, 0)),
            pl.BlockSpec((TSA, 1, 16), lambda i: (i, 0, 0)),
            pl.BlockSpec((C, 3), lambda i: (0, 0)),
            pl.BlockSpec((1, C), lambda i: (0, 0)),
            pl.BlockSpec((1, C), lambda i: (0, 0)),
            pl.BlockSpec((G, C), lambda i: (0, 0)),
        ],
        out_specs=[
            pl.BlockSpec((TSA, K, G), lambda i: (i, 0, 0)),
            pl.BlockSpec((8, 8), lambda i: (0, 0)),
        ],
        out_shape=[
            jax.ShapeDtypeStruct((BN_, K, G), jnp.float32),
            jax.ShapeDtypeStruct((8, 8), jnp.float32),
        ],
    )(g3, t23, Wpe1, ape, cpe, M)


def _k5_body(z_ref, az_ref, cz_ref, wblk_ref, t8_ref, w_ref):
    a = jnp.maximum(z_ref[...] * az_ref[...] + cz_ref[...], 0.0)
    l = jnp.dot(a, wblk_ref[...], preferred_element_type=jnp.float32)
    m8 = l[:, 0:G]
    for kk in range(1, K):
        m8 = jnp.maximum(m8, l[:, kk * G:(kk + 1) * G])
    e = jnp.exp(l - jnp.dot(m8, t8_ref[...], precision=_HP,
                            preferred_element_type=jnp.float32))
    s8 = e[:, 0:G]
    for kk in range(1, K):
        s8 = s8 + e[:, kk * G:(kk + 1) * G]
    w_ref[...] = e / jnp.dot(s8, t8_ref[...], precision=_HP,
                             preferred_element_type=jnp.float32)


def _k5(z2, az, cz, Wblk, T8):
    return pl.pallas_call(
        _k5_body,
        grid=(BN_ // TP,),
        in_specs=[
            pl.BlockSpec((TP, K * G), lambda i: (i, 0)),
            pl.BlockSpec((1, K * G), lambda i: (0, 0)),
            pl.BlockSpec((1, K * G), lambda i: (0, 0)),
            pl.BlockSpec((K * G, K * G), lambda i: (0, 0)),
            pl.BlockSpec((G, K * G), lambda i: (0, 0)),
        ],
        out_specs=pl.BlockSpec((TP, K * G), lambda i: (i, 0)),
        out_shape=jax.ShapeDtypeStruct((BN_, K * G), jnp.float32),
    )(z2, az, cz, Wblk, T8)


def _k6_body(gv_ref, g1_ref, t2_ref, w_ref, wpe1_ref, ape_ref, cpe_ref,
             wpe2_ref, e_ref, wo_ref, yo_ref, st_ref):
    i = pl.program_id(0)
    cen3 = t2_ref[:, 0, 0:3]
    acc = jnp.zeros((TP2, C), jnp.float32)
    for kk in range(K):
        vk = gv_ref[:, kk, :]
        relk = g1_ref[:, kk, 0:3] - cen3
        wk = w_ref[:, kk * G:(kk + 1) * G]
        wek = jnp.dot(wk, e_ref[...], precision=_HP,
                      preferred_element_type=jnp.float32)
        pe1 = lax.dot_general(relk, wpe1_ref[...], (((1,), (1,)), ((), ())),
                              preferred_element_type=jnp.float32)
        h = jnp.maximum(pe1 * ape_ref[...] + cpe_ref[...], 0.0)
        pek = lax.dot_general(h, wpe2_ref[...], (((1,), (1,)), ((), ())),
                              preferred_element_type=jnp.float32)
        acc = acc + (vk + pek) * wek
    yo = lax.dot_general(acc, wo_ref[...], (((1,), (1,)), ((), ())),
                         preferred_element_type=jnp.float32)
    yo_ref[...] = yo
    su = jnp.sum(yo, axis=0, keepdims=True)
    sq = jnp.sum(yo * yo, axis=0, keepdims=True)
    r8 = lax.broadcasted_iota(jnp.int32, (8, 1), 0)
    a0 = (r8 == 0).astype(jnp.float32)
    a1 = (r8 == 1).astype(jnp.float32)
    part = (jnp.dot(a0, su, precision=_HP,
                    preferred_element_type=jnp.float32)
            + jnp.dot(a1, sq, precision=_HP,
                      preferred_element_type=jnp.float32))

    @pl.when(i == 0)
    def _():
        st_ref[...] = jnp.zeros_like(st_ref)

    st_ref[...] += part


def _k6(gv3, g13, t23, w2, Wpe1, ape, cpe, Wpe2, E, Wo):
    return pl.pallas_call(
        _k6_body,
        grid=(BN_ // TP2,),
        in_specs=[
            pl.BlockSpec((TP2, K, C), lambda i: (i, 0, 0)),
            pl.BlockSpec((TP2, K, 128), lambda i: (i, 0, 0)),
            pl.BlockSpec((TP2, 1, 16), lambda i: (i, 0, 0)),
            pl.BlockSpec((TP2, K * G), lambda i: (i, 0)),
            pl.BlockSpec((C, 3), lambda i: (0, 0)),
            pl.BlockSpec((1, C), lambda i: (0, 0)),
            pl.BlockSpec((1, C), lambda i: (0, 0)),
            pl.BlockSpec((C, C), lambda i: (0, 0)),
            pl.BlockSpec((G, C), lambda i: (0, 0)),
            pl.BlockSpec((C, C), lambda i: (0, 0)),
        ],
        out_specs=[
            pl.BlockSpec((TP2, C), lambda i: (i, 0)),
            pl.BlockSpec((8, C), lambda i: (0, 0)),
        ],
        out_shape=[
            jax.ShapeDtypeStruct((BN_, C), jnp.float32),
            jax.ShapeDtypeStruct((8, C), jnp.float32),
        ],
    )(gv3, g13, t23, w2, Wpe1, ape, cpe, Wpe2, E, Wo)


def _k7_body(yo_ref, ao_ref, co_ref, out_ref):
    yo = yo_ref[...]
    r = lax.broadcasted_iota(jnp.int32, (C, C), 0)
    c = lax.broadcasted_iota(jnp.int32, (C, C), 1)
    ident = (r == c).astype(jnp.float32)
    t = lax.dot_general(ident, yo, (((1,), (1,)), ((), ())),
                        precision=_HP,
                        preferred_element_type=jnp.float32)
    out_ref[0] = t * ao_ref[...] + co_ref[...]


def _k7(yoT, ao, co):
    return pl.pallas_call(
        _k7_body,
        grid=(B, N // TN),
        in_specs=[
            pl.BlockSpec((TN, C), lambda b, j: (b * (N // TN) + j, 0)),
            pl.BlockSpec((C, 1), lambda b, j: (0, 0)),
            pl.BlockSpec((C, 1), lambda b, j: (0, 0)),
        ],
        out_specs=pl.BlockSpec((1, C, TN), lambda b, j: (b, 0, j)),
        out_shape=jax.ShapeDtypeStruct((B, C, N), jnp.float32),
    )(yoT, ao, co)


def _bn_affine(s_sum, s_sq, n, gamma, beta):
    mu = s_sum / n
    var = s_sq / n - mu * mu
    inv = 1.0 / jnp.sqrt(var + 1e-5)
    a = gamma * inv
    c = beta - mu * a
    return a, c


@jax.jit
def kernel(x, xyz, Wq, gq, bq, Wk, gk, bk, Wv, Wpe1, gpe, bpe, Wpe2,
           Wwe1, gwe, bwe, Wwe2, Wo, go, bo):
    ptsT = jnp.transpose(xyz, (0, 2, 1))

    yq, yk, vT, st1 = _k1(x, Wq, Wk, Wv)
    idx = _k2(ptsT)
    idxf = idx.reshape(S_)

    aq, cq = _bn_affine(st1[:, 0], st1[:, 1], BN_, gq, bq)
    ak, ck = _bn_affine(st1[:, 2], st1[:, 3], BN_, gk, bk)

    T1, T2 = _k3(yq, yk, ptsT, Wwe1,
                 aq.reshape(C, 1), cq.reshape(C, 1),
                 ak.reshape(C, 1), ck.reshape(C, 1))

    Gv = _sc_gather(vT, idxf, C)
    G1 = _sc_gather(T1, idxf, 128)
    g3 = G1.reshape(BN_, K, 128)
    gv3 = Gv.reshape(BN_, K, C)
    t23 = T2.reshape(BN_, 1, 16)

    st_rel = _k4a(g3, t23)
    mu_rel = st_rel[0, 0:3] / S_
    S_rel = st_rel[1:4, 0:3] / S_
    cov = S_rel - jnp.outer(mu_rel, mu_rel)
    mu_pe1 = Wpe1 @ mu_rel
    var_pe1 = jnp.sum((Wpe1 @ cov) * Wpe1, axis=1)
    inv_pe = 1.0 / jnp.sqrt(var_pe1 + 1e-5)
    ape = (gpe * inv_pe).reshape(1, C)
    cpe = (bpe - mu_pe1 * gpe * inv_pe).reshape(1, C)

    M = Wwe1 @ Wpe2
    z, st_z = _k4(g3, t23, Wpe1, ape, cpe, M)

    az, cz = _bn_affine(st_z[0, :], st_z[1, :], S_, gwe, bwe)
    az128 = jnp.tile(az, K).reshape(1, K * G)
    cz128 = jnp.tile(cz, K).reshape(1, K * G)
    Wblk = jnp.kron(jnp.eye(K, dtype=jnp.float32), Wwe2.T)
    T8 = jnp.kron(jnp.ones((1, K), jnp.float32), jnp.eye(G, dtype=jnp.float32))

    z2 = z.reshape(BN_, K * G)
    w2 = _k5(z2, az128, cz128, Wblk, T8)

    E = jnp.kron(jnp.eye(G, dtype=jnp.float32),
                 jnp.ones((1, C // G), jnp.float32))
    yoT, st_o = _k6(gv3, g3, t23, w2, Wpe1, ape, cpe, Wpe2, E, Wo)

    ao, co = _bn_affine(st_o[0, :], st_o[1, :], BN_, go, bo)
    return _k7(yoT, ao.reshape(C, 1), co.reshape(C, 1))


# speedup vs baseline: 10.2827x; 1.0290x over previous
import functools

import jax
import jax.numpy as jnp
from jax import lax
from jax.experimental import pallas as pl
from jax.experimental.pallas import tpu as pltpu
from jax.experimental.pallas import tpu_sc as plsc

_HP = lax.Precision.HIGHEST

B, C, N, G, K = 2, 256, 4096, 8, 16
BN_ = B * N
S_ = BN_ * K
W_ = C + 128
NW = 32
SPW = S_ // NW
CH = 128

TN = 512
TQ = 512
TSA = 512
TP = 1024
TP2 = 256


def _first(*ids):
    f = ids[0] == 0
    for i in ids[1:]:
        f = jnp.logical_and(f, i == 0)
    return f


def _k1_body(x_ref, wq_ref, wk_ref, wv_ref, yq_ref, yk_ref, vt_ref, st_ref):
    b = pl.program_id(0)
    j = pl.program_id(1)
    x_t = x_ref[0]
    yq = jnp.dot(wq_ref[...], x_t, preferred_element_type=jnp.float32)
    yk = jnp.dot(wk_ref[...], x_t, preferred_element_type=jnp.float32)
    vt = lax.dot_general(x_t, wv_ref[...], (((0,), (1,)), ((), ())),
                         preferred_element_type=jnp.float32)
    yq_ref[0] = yq
    yk_ref[0] = yk
    vt_ref[...] = vt
    col = lax.broadcasted_iota(jnp.int32, (TN, 8), 1)
    p0 = (col == 0).astype(jnp.float32)
    p1 = (col == 1).astype(jnp.float32)
    p2 = (col == 2).astype(jnp.float32)
    p3 = (col == 3).astype(jnp.float32)
    part = (jnp.dot(yq, p0, precision=_HP, preferred_element_type=jnp.float32)
            + jnp.dot(yq * yq, p1, precision=_HP,
                      preferred_element_type=jnp.float32)
            + jnp.dot(yk, p2, precision=_HP,
                      preferred_element_type=jnp.float32)
            + jnp.dot(yk * yk, p3, precision=_HP,
                      preferred_element_type=jnp.float32))

    @pl.when(_first(b, j))
    def _():
        st_ref[...] = jnp.zeros_like(st_ref)

    st_ref[...] += part


def _k1(x, Wq, Wk, Wv):
    return pl.pallas_call(
        _k1_body,
        grid=(B, N // TN),
        in_specs=[
            pl.BlockSpec((1, C, TN), lambda b, j: (b, 0, j)),
            pl.BlockSpec((C, C), lambda b, j: (0, 0)),
            pl.BlockSpec((C, C), lambda b, j: (0, 0)),
            pl.BlockSpec((C, C), lambda b, j: (0, 0)),
        ],
        out_specs=[
            pl.BlockSpec((1, C, TN), lambda b, j: (b, 0, j)),
            pl.BlockSpec((1, C, TN), lambda b, j: (b, 0, j)),
            pl.BlockSpec((TN, C), lambda b, j: (b * (N // TN) + j, 0)),
            pl.BlockSpec((C, 8), lambda b, j: (0, 0)),
        ],
        out_shape=[
            jax.ShapeDtypeStruct((B, C, N), jnp.float32),
            jax.ShapeDtypeStruct((B, C, N), jnp.float32),
            jax.ShapeDtypeStruct((BN_, C), jnp.float32),
            jax.ShapeDtypeStruct((C, 8), jnp.float32),
        ],
    )(x, Wq, Wk, Wv)


def _k2_body(ptsq_ref, ptsa_ref, idx_ref):
    b = pl.program_id(0)
    qt = ptsq_ref[0]
    ap = ptsa_ref[0]
    sq_q = jnp.sum(qt * qt, axis=1, keepdims=True)
    sq_ac = jnp.sum(ap * ap, axis=1, keepdims=True)
    sq_a = lax.dot_general(jnp.ones((1, 1), jnp.float32), sq_ac,
                           (((1,), (1,)), ((), ())),
                           precision=lax.Precision.HIGHEST,
                           preferred_element_type=jnp.float32)
    cross = lax.dot_general(qt.astype(jnp.bfloat16),
                            ap.astype(jnp.bfloat16),
                            (((1,), (1,)), ((), ())),
                            preferred_element_type=jnp.float32)
    d = sq_q + sq_a - 2.0 * cross
    iota_n = lax.broadcasted_iota(jnp.int32, (TQ, N), 1)
    lane16 = lax.broadcasted_iota(jnp.int32, (TQ, K), 1)
    acc = jnp.zeros((TQ, K), jnp.int32)
    big = jnp.float32(jnp.inf)
    for j in range(K):
        am = jnp.argmin(d, axis=1, keepdims=True).astype(jnp.int32)
        acc = jnp.where(lane16 == j, am + b * N, acc)
        d = jnp.where(iota_n == am, big, d)
    idx_ref[...] = acc


def _k2(ptsT):
    return pl.pallas_call(
        _k2_body,
        grid=(B, N // TQ),
        in_specs=[
            pl.BlockSpec((1, TQ, 3), lambda b, j: (b, j, 0)),
            pl.BlockSpec((1, N, 3), lambda b, j: (b, 0, 0)),
        ],
        out_specs=pl.BlockSpec((TQ, K), lambda b, j: (b * (N // TQ) + j, 0)),
        out_shape=jax.ShapeDtypeStruct((BN_, K), jnp.int32),
    )(ptsT, ptsT)


def _k3_body(yq_ref, yk_ref, xyzt_ref, wwe1_ref, aq_ref, cq_ref,
             ak_ref, ck_ref, t1_ref, t2_ref):
    q = jnp.maximum(yq_ref[0] * aq_ref[...] + cq_ref[...], 0.0)
    k = jnp.maximum(yk_ref[0] * ak_ref[...] + ck_ref[...], 0.0)
    zq = lax.dot_general(q, wwe1_ref[...], (((0,), (1,)), ((), ())),
                         preferred_element_type=jnp.float32)
    zk = lax.dot_general(k, wwe1_ref[...], (((0,), (1,)), ((), ())),
                         preferred_element_type=jnp.float32)
    xyz_t = xyzt_ref[0]
    r3 = lax.broadcasted_iota(jnp.int32, (3, 128), 0)
    c3 = lax.broadcasted_iota(jnp.int32, (3, 128), 1)
    p3 = (r3 == c3).astype(jnp.float32)
    r8 = lax.broadcasted_iota(jnp.int32, (G, 128), 0)
    c8 = lax.broadcasted_iota(jnp.int32, (G, 128), 1)
    p8 = (c8 == r8 + 3).astype(jnp.float32)
    xp = jnp.dot(xyz_t, p3, precision=_HP, preferred_element_type=jnp.float32)
    t1_ref[...] = xp + jnp.dot(zk, p8, precision=_HP,
                               preferred_element_type=jnp.float32)
    t2_ref[...] = (xp + jnp.dot(zq, p8, precision=_HP,
                                preferred_element_type=jnp.float32))[:, 0:16]


def _k3(yq, yk, xyzT, Wwe1, aq, cq, ak, ck):
    return pl.pallas_call(
        _k3_body,
        grid=(B, N // TN),
        in_specs=[
            pl.BlockSpec((1, C, TN), lambda b, j: (b, 0, j)),
            pl.BlockSpec((1, C, TN), lambda b, j: (b, 0, j)),
            pl.BlockSpec((1, TN, 3), lambda b, j: (b, j, 0)),
            pl.BlockSpec((G, C), lambda b, j: (0, 0)),
            pl.BlockSpec((C, 1), lambda b, j: (0, 0)),
            pl.BlockSpec((C, 1), lambda b, j: (0, 0)),
            pl.BlockSpec((C, 1), lambda b, j: (0, 0)),
            pl.BlockSpec((C, 1), lambda b, j: (0, 0)),
        ],
        out_specs=[
            pl.BlockSpec((TN, 128), lambda b, j: (b * (N // TN) + j, 0)),
            pl.BlockSpec((TN, 16), lambda b, j: (b * (N // TN) + j, 0)),
        ],
        out_shape=[
            jax.ShapeDtypeStruct((BN_, 128), jnp.float32),
            jax.ShapeDtypeStruct((BN_, 16), jnp.float32),
        ],
    )(yq, yk, xyzT, Wwe1, aq, cq, ak, ck)


def _sc_gather(table, idxf, width):
    mesh = plsc.VectorSubcoreMesh(core_axis_name="c", subcore_axis_name="s")

    @functools.partial(
        pl.kernel,
        out_type=jax.ShapeDtypeStruct((S_, width), jnp.float32),
        mesh=mesh,
        scratch_types=[
            pltpu.VMEM((2, CH), jnp.int32),
            pltpu.VMEM((2, CH, width), jnp.float32),
            pltpu.SemaphoreType.DMA,
            pltpu.SemaphoreType.DMA,
        ],
    )
    def scg(t_hbm, idx_hbm, out_hbm, idx2, rows2, sem0, sem1):
        wid = lax.axis_index("s") * 2 + lax.axis_index("c")
        base_s = wid * SPW
        sems = [sem0, sem1]

        def body_c(j, _):
            handles = []
            for bb in range(2):
                off = base_s + (2 * j + bb) * CH
                pltpu.sync_copy(idx_hbm.at[pl.ds(off, CH)], idx2.at[bb])
                handles.append(
                    pltpu.async_copy(t_hbm.at[idx2.at[bb]], rows2.at[bb],
                                     sems[bb]))
            for bb in range(2):
                off = base_s + (2 * j + bb) * CH
                handles[bb].wait()
                pltpu.sync_copy(rows2.at[bb], out_hbm.at[pl.ds(off, CH)])
            return 0

        lax.fori_loop(0, SPW // (2 * CH), body_c, 0)

    return scg(table, idxf)


def _k4a_body(g_ref, t2_ref, st_ref):
    i = pl.program_id(0)
    cen = t2_ref[:, 0, 0:3]
    rel_all = jnp.concatenate(
        [g_ref[:, kk, 0:3] - cen for kk in range(K)], axis=0)
    su = lax.dot_general(jnp.ones((8, TSA * K), jnp.float32), rel_all,
                         (((1,), (0,)), ((), ())),
                         precision=_HP,
                         preferred_element_type=jnp.float32)[0:1]
    outer = lax.dot_general(rel_all, rel_all, (((0,), (0,)), ((), ())),
                            precision=_HP,
                            preferred_element_type=jnp.float32)
    r8 = lax.broadcasted_iota(jnp.int32, (8, 1), 0)
    a0 = (r8 == 0).astype(jnp.float32)
    rb = lax.broadcasted_iota(jnp.int32, (8, 3), 0)
    cb = lax.broadcasted_iota(jnp.int32, (8, 3), 1)
    bsel = (rb == cb + 1).astype(jnp.float32)
    r38 = lax.broadcasted_iota(jnp.int32, (3, 8), 0)
    c38 = lax.broadcasted_iota(jnp.int32, (3, 8), 1)
    p38 = (r38 == c38).astype(jnp.float32)
    part = jnp.dot(jnp.dot(a0, su, precision=_HP,
                           preferred_element_type=jnp.float32)
                   + jnp.dot(bsel, outer, precision=_HP,
                             preferred_element_type=jnp.float32),
                   p38, precision=_HP,
                   preferred_element_type=jnp.float32)

    @pl.when(i == 0)
    def _():
        st_ref[...] = jnp.zeros_like(st_ref)

    st_ref[...] += part


def _k4a(g3, t23):
    return pl.pallas_call(
        _k4a_body,
        grid=(BN_ // TSA,),
        in_specs=[
            pl.BlockSpec((TSA, K, 128), lambda i: (i, 0, 0)),
            pl.BlockSpec((TSA, 1, 16), lambda i: (i, 0, 0)),
        ],
        out_specs=pl.BlockSpec((8, 8), lambda i: (0, 0)),
        out_shape=jax.ShapeDtypeStruct((8, 8), jnp.float32),
    )(g3, t23)


def _k4_body(g_ref, t2_ref, wpe1_ref, ape_ref, cpe_ref, m_ref, z_ref, st_ref):
    i = pl.program_id(0)
    cen3 = t2_ref[:, 0, 0:3]
    cen8 = t2_ref[:, 0, 3:11]
    rel_all = jnp.concatenate(
        [g_ref[:, kk, 0:3] - cen3 for kk in range(K)], axis=0)
    zdiff_all = jnp.concatenate(
        [g_ref[:, kk, 3:11] - cen8 for kk in range(K)], axis=0)
    pe1 = lax.dot_general(rel_all, wpe1_ref[...], (((1,), (1,)), ((), ())),
                          preferred_element_type=jnp.float32)
    h = jnp.maximum(pe1 * ape_ref[...] + cpe_ref[...], 0.0)
    zpe = lax.dot_general(h, m_ref[...], (((1,), (1,)), ((), ())),
                          preferred_element_type=jnp.float32)
    z_all = zdiff_all + zpe
    for kk in range(K):
        z_ref[:, kk, :] = z_all[kk * TSA:(kk + 1) * TSA, :]
    su = jnp.sum(z_all, axis=0, keepdims=True)
    sq = jnp.sum(z_all * z_all, axis=0, keepdims=True)
    r8 = lax.broadcasted_iota(jnp.int32, (8, 1), 0)
    a0 = (r8 == 0).astype(jnp.float32)
    a1 = (r8 == 1).astype(jnp.float32)
    part = (jnp.dot(a0, su, precision=_HP,
                    preferred_element_type=jnp.float32)
            + jnp.dot(a1, sq, precision=_HP,
                      preferred_element_type=jnp.float32))

    @pl.when(i == 0)
    def _():
        st_ref[...] = jnp.zeros_like(st_ref)

    st_ref[...] += part


def _k4(g3, t23, Wpe1, ape, cpe, M):
    return pl.pallas_call(
        _k4_body,
        grid=(BN_ // TSA,),
        in_specs=[
            pl.BlockSpec((TSA, K, 128), lambda i: (i, 0, 0)),
            pl.BlockSpec((TSA, 1, 16), lambda i: (i, 0, 0)),
            pl.BlockSpec((C, 3), lambda i: (0, 0)),
            pl.BlockSpec((1, C), lambda i: (0, 0)),
            pl.BlockSpec((1, C), lambda i: (0, 0)),
            pl.BlockSpec((G, C), lambda i: (0, 0)),
        ],
        out_specs=[
            pl.BlockSpec((TSA, K, G), lambda i: (i, 0, 0)),
            pl.BlockSpec((8, 8), lambda i: (0, 0)),
        ],
        out_shape=[
            jax.ShapeDtypeStruct((BN_, K, G), jnp.float32),
            jax.ShapeDtypeStruct((8, 8), jnp.float32),
        ],
    )(g3, t23, Wpe1, ape, cpe, M)


def _k5_body(z_ref, az_ref, cz_ref, wblk_ref, t8_ref, w_ref):
    a = jnp.maximum(z_ref[...] * az_ref[...] + cz_ref[...], 0.0)
    l = jnp.dot(a, wblk_ref[...], preferred_element_type=jnp.float32)
    m8 = l[:, 0:G]
    for kk in range(1, K):
        m8 = jnp.maximum(m8, l[:, kk * G:(kk + 1) * G])
    e = jnp.exp(l - jnp.dot(m8, t8_ref[...], precision=_HP,
                            preferred_element_type=jnp.float32))
    s8 = e[:, 0:G]
    for kk in range(1, K):
        s8 = s8 + e[:, kk * G:(kk + 1) * G]
    w_ref[...] = e / jnp.dot(s8, t8_ref[...], precision=_HP,
                             preferred_element_type=jnp.float32)


def _k5(z2, az, cz, Wblk, T8):
    return pl.pallas_call(
        _k5_body,
        grid=(BN_ // TP,),
        in_specs=[
            pl.BlockSpec((TP, K * G), lambda i: (i, 0)),
            pl.BlockSpec((1, K * G), lambda i: (0, 0)),
            pl.BlockSpec((1, K * G), lambda i: (0, 0)),
            pl.BlockSpec((K * G, K * G), lambda i: (0, 0)),
            pl.BlockSpec((G, K * G), lambda i: (0, 0)),
        ],
        out_specs=pl.BlockSpec((TP, K * G), lambda i: (i, 0)),
        out_shape=jax.ShapeDtypeStruct((BN_, K * G), jnp.float32),
    )(z2, az, cz, Wblk, T8)


def _k6_body(gv_ref, g1_ref, t2_ref, w_ref, wpe1_ref, ape_ref, cpe_ref,
             wpe2_ref, e_ref, wo_ref, yo_ref, st_ref):
    i = pl.program_id(0)
    cen3 = t2_ref[:, 0, 0:3]
    rel_all = jnp.concatenate(
        [g1_ref[:, kk, 0:3] - cen3 for kk in range(K)], axis=0)
    w_all = jnp.concatenate(
        [w_ref[:, kk * G:(kk + 1) * G] for kk in range(K)], axis=0)
    wek = jnp.dot(w_all, e_ref[...], precision=_HP,
                  preferred_element_type=jnp.float32)
    pe1 = lax.dot_general(rel_all, wpe1_ref[...], (((1,), (1,)), ((), ())),
                          preferred_element_type=jnp.float32)
    h = jnp.maximum(pe1 * ape_ref[...] + cpe_ref[...], 0.0)
    pek = lax.dot_general(h, wpe2_ref[...], (((1,), (1,)), ((), ())),
                          preferred_element_type=jnp.float32)
    acc = jnp.zeros((TP2, C), jnp.float32)
    for kk in range(K):
        sl = slice(kk * TP2, (kk + 1) * TP2)
        acc = acc + (gv_ref[:, kk, :] + pek[sl, :]) * wek[sl, :]
    yo = lax.dot_general(acc, wo_ref[...], (((1,), (1,)), ((), ())),
                         preferred_element_type=jnp.float32)
    yo_ref[...] = yo
    su = jnp.sum(yo, axis=0, keepdims=True)
    sq = jnp.sum(yo * yo, axis=0, keepdims=True)
    r8 = lax.broadcasted_iota(jnp.int32, (8, 1), 0)
    a0 = (r8 == 0).astype(jnp.float32)
    a1 = (r8 == 1).astype(jnp.float32)
    part = (jnp.dot(a0, su, precision=_HP,
                    preferred_element_type=jnp.float32)
            + jnp.dot(a1, sq, precision=_HP,
                      preferred_element_type=jnp.float32))

    @pl.when(i == 0)
    def _():
        st_ref[...] = jnp.zeros_like(st_ref)

    st_ref[...] += part


def _k6(gv3, g13, t23, w2, Wpe1, ape, cpe, Wpe2, E, Wo):
    return pl.pallas_call(
        _k6_body,
        grid=(BN_ // TP2,),
        in_specs=[
            pl.BlockSpec((TP2, K, C), lambda i: (i, 0, 0)),
            pl.BlockSpec((TP2, K, 128), lambda i: (i, 0, 0)),
            pl.BlockSpec((TP2, 1, 16), lambda i: (i, 0, 0)),
            pl.BlockSpec((TP2, K * G), lambda i: (i, 0)),
            pl.BlockSpec((C, 3), lambda i: (0, 0)),
            pl.BlockSpec((1, C), lambda i: (0, 0)),
            pl.BlockSpec((1, C), lambda i: (0, 0)),
            pl.BlockSpec((C, C), lambda i: (0, 0)),
            pl.BlockSpec((G, C), lambda i: (0, 0)),
            pl.BlockSpec((C, C), lambda i: (0, 0)),
        ],
        out_specs=[
            pl.BlockSpec((TP2, C), lambda i: (i, 0)),
            pl.BlockSpec((8, C), lambda i: (0, 0)),
        ],
        out_shape=[
            jax.ShapeDtypeStruct((BN_, C), jnp.float32),
            jax.ShapeDtypeStruct((8, C), jnp.float32),
        ],
    )(gv3, g13, t23, w2, Wpe1, ape, cpe, Wpe2, E, Wo)


def _k7_body(yo_ref, ao_ref, co_ref, out_ref):
    yo = yo_ref[...]
    r = lax.broadcasted_iota(jnp.int32, (C, C), 0)
    c = lax.broadcasted_iota(jnp.int32, (C, C), 1)
    ident = (r == c).astype(jnp.float32)
    t = lax.dot_general(ident, yo, (((1,), (1,)), ((), ())),
                        precision=_HP,
                        preferred_element_type=jnp.float32)
    out_ref[0] = t * ao_ref[...] + co_ref[...]


def _k7(yoT, ao, co):
    return pl.pallas_call(
        _k7_body,
        grid=(B, N // TN),
        in_specs=[
            pl.BlockSpec((TN, C), lambda b, j: (b * (N // TN) + j, 0)),
            pl.BlockSpec((C, 1), lambda b, j: (0, 0)),
            pl.BlockSpec((C, 1), lambda b, j: (0, 0)),
        ],
        out_specs=pl.BlockSpec((1, C, TN), lambda b, j: (b, 0, j)),
        out_shape=jax.ShapeDtypeStruct((B, C, N), jnp.float32),
    )(yoT, ao, co)


def _bn_affine(s_sum, s_sq, n, gamma, beta):
    mu = s_sum / n
    var = s_sq / n - mu * mu
    inv = 1.0 / jnp.sqrt(var + 1e-5)
    a = gamma * inv
    c = beta - mu * a
    return a, c


@jax.jit
def kernel(x, xyz, Wq, gq, bq, Wk, gk, bk, Wv, Wpe1, gpe, bpe, Wpe2,
           Wwe1, gwe, bwe, Wwe2, Wo, go, bo):
    ptsT = jnp.transpose(xyz, (0, 2, 1))

    yq, yk, vT, st1 = _k1(x, Wq, Wk, Wv)
    idx = _k2(ptsT)
    idxf = idx.reshape(S_)

    aq, cq = _bn_affine(st1[:, 0], st1[:, 1], BN_, gq, bq)
    ak, ck = _bn_affine(st1[:, 2], st1[:, 3], BN_, gk, bk)

    T1, T2 = _k3(yq, yk, ptsT, Wwe1,
                 aq.reshape(C, 1), cq.reshape(C, 1),
                 ak.reshape(C, 1), ck.reshape(C, 1))

    Gv = _sc_gather(vT, idxf, C)
    G1 = _sc_gather(T1, idxf, 128)
    g3 = G1.reshape(BN_, K, 128)
    gv3 = Gv.reshape(BN_, K, C)
    t23 = T2.reshape(BN_, 1, 16)

    st_rel = _k4a(g3, t23)
    mu_rel = st_rel[0, 0:3] / S_
    S_rel = st_rel[1:4, 0:3] / S_
    cov = S_rel - jnp.outer(mu_rel, mu_rel)
    mu_pe1 = Wpe1 @ mu_rel
    var_pe1 = jnp.sum((Wpe1 @ cov) * Wpe1, axis=1)
    inv_pe = 1.0 / jnp.sqrt(var_pe1 + 1e-5)
    ape = (gpe * inv_pe).reshape(1, C)
    cpe = (bpe - mu_pe1 * gpe * inv_pe).reshape(1, C)

    M = Wwe1 @ Wpe2
    z, st_z = _k4(g3, t23, Wpe1, ape, cpe, M)

    az, cz = _bn_affine(st_z[0, :], st_z[1, :], S_, gwe, bwe)
    az128 = jnp.tile(az, K).reshape(1, K * G)
    cz128 = jnp.tile(cz, K).reshape(1, K * G)
    Wblk = jnp.kron(jnp.eye(K, dtype=jnp.float32), Wwe2.T)
    T8 = jnp.kron(jnp.ones((1, K), jnp.float32), jnp.eye(G, dtype=jnp.float32))

    z2 = z.reshape(BN_, K * G)
    w2 = _k5(z2, az128, cz128, Wblk, T8)

    E = jnp.kron(jnp.eye(G, dtype=jnp.float32),
                 jnp.ones((1, C // G), jnp.float32))
    yoT, st_o = _k6(gv3, g3, t23, w2, Wpe1, ape, cpe, Wpe2, E, Wo)

    ao, co = _bn_affine(st_o[0, :], st_o[1, :], BN_, go, bo)
    return _k7(yoT, ao.reshape(C, 1), co.reshape(C, 1))
